# baseline (speedup 1.0000x reference)
import functools

import jax
import jax.numpy as jnp
from jax import lax
from jax.experimental import pallas as pl
from jax.experimental.pallas import tpu as pltpu
from jax.experimental.pallas import tpu_sc as plsc

N = 10000
NF = 128
EF = 5
MPAD = 4096
NE = 24576
NCORE = 2
NWORK = 16 * NCORE
ROWS = 320
NSLOT = 1024
CH = 128
NCHS = NSLOT // CH
NE2 = NWORK * NSLOT
BI = 8
BJ = 128
NBLK = N // BI
NROW = 1000
NOUT = NWORK * ROWS


def _silu(x):
    return x * (1.0 / (1.0 + jnp.exp(-x)))


def _lnorm(x, g, b):
    m = jnp.mean(x, axis=-1, keepdims=True)
    c = x - m
    v = jnp.mean(c * c, axis=-1, keepdims=True)
    return c * lax.rsqrt(v + 1e-5) * g + b


def _mmT(x, w):
    return lax.dot_general(x, w, (((x.ndim - 1,), (1,)), ((), ())),
                           preferred_element_type=jnp.float32)


def _emb_body(h_ref, w0_ref, b0_ref, w1_ref, b1_ref, o_ref):
    t = _silu(_mmT(h_ref[...], w0_ref[...]) + b0_ref[...])
    o_ref[...] = _mmT(t, w1_ref[...]) + b1_ref[...]


def _emb_call(h, w0, b0, w1, b1):
    wspec = pl.BlockSpec((NF, NF), lambda i: (0, 0))
    bspec = pl.BlockSpec((1, NF), lambda i: (0, 0))
    return pl.pallas_call(
        _emb_body,
        grid=(N // NROW,),
        in_specs=[pl.BlockSpec((NROW, NF), lambda i: (i, 0)),
                  wspec, bspec, wspec, bspec],
        out_specs=pl.BlockSpec((NROW, NF), lambda i: (i, 0)),
        out_shape=jax.ShapeDtypeStruct((N, NF), jnp.float32),
    )(h, w0, b0.reshape(1, NF), w1, b1.reshape(1, NF))


def _tq_body(bt_ref, wc_ref, o_ref):
    o_ref[...] = lax.dot_general(
        bt_ref[...], wc_ref[...][0], (((1,), (1,)), ((), ())),
        preferred_element_type=jnp.float32)[None]


def _tq_call(bte, wc3):
    nl = wc3.shape[0]
    blk = 2048
    return pl.pallas_call(
        _tq_body,
        grid=(nl, NE2 // blk),
        in_specs=[pl.BlockSpec((blk, EF), lambda l, j: (j, 0)),
                  pl.BlockSpec((1, NF, EF), lambda l, j: (l, 0, 0))],
        out_specs=pl.BlockSpec((1, blk, NF), lambda l, j: (l, j, 0)),
        out_shape=jax.ShapeDtypeStruct((nl, NE2, NF), jnp.float32),
    )(bte, wc3)


def _pre_body(h_ref, g_ref, b_ref, wa_ref, wb_ref, cv_ref, g1_ref, b1_ref,
              hn_ref, u_ref, v_ref, sact_ref):
    hn = _lnorm(h_ref[...], g_ref[...], b_ref[...])
    u = _mmT(hn, wa_ref[...]) + cv_ref[...]
    v = _mmT(hn, wb_ref[...])
    hn_ref[...] = hn
    u_ref[...] = u
    v_ref[...] = v
    sact_ref[...] = _silu(_lnorm(u + v, g1_ref[...], b1_ref[...]))


def _pre_call(h, g, b, wa, wb, cv, g1, b1):
    wspec = pl.BlockSpec((NF, NF), lambda i: (0, 0))
    bspec = pl.BlockSpec((1, NF), lambda i: (0, 0))
    rspec = pl.BlockSpec((NROW, NF), lambda i: (i, 0))
    sd = jax.ShapeDtypeStruct((N, NF), jnp.float32)
    return pl.pallas_call(
        _pre_body,
        grid=(N // NROW,),
        in_specs=[rspec, bspec, bspec, wspec, wspec, bspec, bspec, bspec],
        out_specs=[rspec, rspec, rspec, rspec],
        out_shape=[sd, sd, sd, sd],
    )(h, g.reshape(1, NF), b.reshape(1, NF), wa, wb,
      cv.reshape(1, NF), g1.reshape(1, NF), b1.reshape(1, NF))


def _dense_body(jlo_ref, nch_ref, u_ref, s_ref, e_ref, g1_ref, b1_ref,
                vfull_ref, out_ref):
    i = pl.program_id(0)
    j0 = jlo_ref[i]
    nch = nch_ref[i]
    ub = u_ref[...][:, None, :]
    srow = s_ref[...]
    erow = e_ref[...]
    g1 = g1_ref[...][None]
    b1 = b1_ref[...][None]

    def chunk(k, acc):
        jb = j0 + k * BJ
        vblk = vfull_ref[pl.ds(jb, BJ), :]
        z = ub + vblk[None, :, :]
        m = jnp.mean(z, axis=-1, keepdims=True)
        c = z - m
        var = jnp.mean(c * c, axis=-1, keepdims=True)
        a = _silu(c * lax.rsqrt(var + 1e-5) * g1 + b1)
        jid3 = jb + lax.broadcasted_iota(jnp.int32, (BI, BJ, 1), 1)
        msk3 = (jid3 >= srow[:, None, :]) & (jid3 < erow[:, None, :])
        a = jnp.where(msk3, a, 0.0)
        return acc + jnp.sum(a, axis=1)

    out_ref[...] = lax.fori_loop(0, nch, chunk,
                                 jnp.zeros((BI, NF), jnp.float32))


def _dense_call(jlo, nch, u, starts2, ends2, g1, b1, vpad):
    grid_spec = pltpu.PrefetchScalarGridSpec(
        num_scalar_prefetch=2,
        grid=(NBLK,),
        in_specs=[pl.BlockSpec((BI, NF), lambda i, *_: (i, 0)),
                  pl.BlockSpec((BI, 1), lambda i, *_: (i, 0)),
                  pl.BlockSpec((BI, 1), lambda i, *_: (i, 0)),
                  pl.BlockSpec((1, NF), lambda i, *_: (0, 0)),
                  pl.BlockSpec((1, NF), lambda i, *_: (0, 0)),
                  pl.BlockSpec(vpad.shape, lambda i, *_: (0, 0))],
        out_specs=pl.BlockSpec((BI, NF), lambda i, *_: (i, 0)),
    )
    return pl.pallas_call(
        _dense_body,
        grid_spec=grid_spec,
        out_shape=jax.ShapeDtypeStruct((N, NF), jnp.float32),
    )(jlo, nch, u, starts2, ends2, g1.reshape(1, NF), b1.reshape(1, NF), vpad)


def _post_body(hn_ref, sden_ref, sact_ref, acc_ref, cnt_ref, w1_ref, b1_ref,
               wna_ref, wnb_ref, bn0_ref, gn_ref, bn_ref, wn1_ref, bn1_ref,
               out_ref):
    atot = sden_ref[...] - sact_ref[...] + acc_ref[...]
    agg = _mmT(atot, w1_ref[...]) + cnt_ref[...] * b1_ref[...]
    hn = hn_ref[...]
    a = _mmT(hn, wna_ref[...]) + _mmT(agg, wnb_ref[...]) + bn0_ref[...]
    a = _silu(_lnorm(a, gn_ref[...], bn_ref[...]))
    out_ref[...] = hn + _mmT(a, wn1_ref[...]) + bn1_ref[...]


def _post_call(hn, sden, sact, acc, cnt, w1, b1, wna, wnb, bn0, gn, bn,
               wn1, bn1):
    wspec = pl.BlockSpec((NF, NF), lambda i: (0, 0))
    bspec = pl.BlockSpec((1, NF), lambda i: (0, 0))
    rspec = pl.BlockSpec((NROW, NF), lambda i: (i, 0))
    return pl.pallas_call(
        _post_body,
        grid=(N // NROW,),
        in_specs=[rspec, rspec, rspec, rspec,
                  pl.BlockSpec((NROW, 1), lambda i: (i, 0)),
                  wspec, bspec, wspec, wspec, bspec, bspec, bspec,
                  wspec, bspec],
        out_specs=rspec,
        out_shape=jax.ShapeDtypeStruct((N, NF), jnp.float32),
    )(hn, sden, sact, acc, cnt, w1, b1.reshape(1, NF), wna, wnb,
      bn0.reshape(1, NF), gn.reshape(1, NF), bn.reshape(1, NF),
      wn1, bn1.reshape(1, NF))


def _allsum16(x):
    for sh in (8, 4, 2, 1):
        idx = (lax.iota(jnp.int32, 16) ^ sh)[:, None]
        g = lax.gather(
            x, idx,
            lax.GatherDimensionNumbers(offset_dims=(),
                                       collapsed_slice_dims=(0,),
                                       start_index_map=(0,)),
            (1,), mode=lax.GatherScatterMode.PROMISE_IN_BOUNDS)
        x = x + g
    return x


def _sc_edges_body(u_hbm, v_hbm, tq_hbm, rs_hbm, gn_hbm, cn_hbm, sgb_hbm,
                   lnp_hbm, zer_hbm, out_hbm,
                   rsv, gnv, cnv, urows, vrows, tqv, sgv, lnv, obuf, table,
                   sem1, sem2):
    cid = lax.axis_index("c")
    sid = lax.axis_index("s")
    wid = sid * NCORE + cid
    base = wid * NSLOT

    pltpu.sync_copy(rs_hbm.at[wid], rsv)
    pltpu.sync_copy(gn_hbm.at[wid], gnv)
    pltpu.sync_copy(cn_hbm.at[wid], cnv)
    pltpu.sync_copy(lnp_hbm, lnv)
    pltpu.sync_copy(zer_hbm, table)

    def chunk(k, c0):
        cp1 = pltpu.async_copy(u_hbm.at[gnv.at[k]], urows, sem1)
        cp2 = pltpu.async_copy(v_hbm.at[cnv.at[k]], vrows, sem2)
        pltpu.sync_copy(tq_hbm.at[pl.ds(base + k * CH, CH)], tqv)
        pltpu.sync_copy(sgb_hbm.at[pl.ds(base + k * CH, CH)], sgv)
        cp1.wait()
        cp2.wait()

        def comp(g):
            for e16 in range(16):
                e = g * 16 + e16
                zq = [urows[e, pl.ds(16 * q, 16)]
                      + vrows[e, pl.ds(16 * q, 16)]
                      + tqv[e, pl.ds(16 * q, 16)] for q in range(8)]
                s = zq[0]
                for q in range(1, 8):
                    s = s + zq[q]
                m = _allsum16(s) * (1.0 / 128.0)
                cq = [zq[q] - m for q in range(8)]
                ss = cq[0] * cq[0]
                for q in range(1, 8):
                    ss = ss + cq[q] * cq[q]
                xv = _allsum16(ss) * (1.0 / 128.0) + 1e-5
                ib = lax.bitcast_convert_type(xv, jnp.int32)
                y = lax.bitcast_convert_type(
                    jnp.int32(0x5F3759DF) - (ib >> 1), jnp.float32)
                xh = xv * 0.5
                for _ in range(3):
                    y = y * (1.5 - xh * y * y)
                sg = sgv[e, :]
                for q in range(8):
                    sl = pl.ds(16 * q, 16)
                    yn = cq[q] * y * lnv[0, sl] + lnv[1, sl]
                    sil = yn * (1.0 / (1.0 + jnp.exp(-yn)))
                    obuf[e, sl] = sil * sg

        plsc.parallel_loop(0, CH // 16)(comp)

        def accum(g, c1):
            r16 = rsv[k, pl.ds(16 * g, 16)]
            for e16 in range(16):
                e = g * 16 + e16
                row = r16[e16]
                for q in range(8):
                    sl = pl.ds(16 * q, 16)
                    table[row, sl] = table[row, sl] + obuf[e, sl]
            return c1

        lax.fori_loop(0, CH // 16, accum, 0)
        return c0

    lax.fori_loop(0, NCHS, chunk, 0)
    pltpu.sync_copy(table, out_hbm.at[pl.ds(wid * ROWS, ROWS)])


@functools.cache
def _make_sc_edges():
    return functools.partial(
        pl.kernel,
        mesh=plsc.VectorSubcoreMesh(core_axis_name="c", subcore_axis_name="s",
                                    num_cores=NCORE),
        compiler_params=pltpu.CompilerParams(needs_layout_passes=False),
        out_type=jax.ShapeDtypeStruct((NOUT, NF), jnp.float32),
        scratch_types=[
            pltpu.VMEM((NCHS, CH), jnp.int32),
            pltpu.VMEM((NCHS, CH), jnp.int32),
            pltpu.VMEM((NCHS, CH), jnp.int32),
            pltpu.VMEM((CH, NF), jnp.float32),
            pltpu.VMEM((CH, NF), jnp.float32),
            pltpu.VMEM((CH, NF), jnp.float32),
            pltpu.VMEM((CH, 16), jnp.float32),
            pltpu.VMEM((2, NF), jnp.float32),
            pltpu.VMEM((CH, NF), jnp.float32),
            pltpu.VMEM((ROWS, NF), jnp.float32),
            pltpu.SemaphoreType.DMA,
            pltpu.SemaphoreType.DMA,
        ],
    )(_sc_edges_body)


def _sc_edges(*args):
    return _make_sc_edges()(*args)


def kernel(x, h, batch_mask, covalent_bonds, bond_types, mol_feats, params):
    nb = covalent_bonds.shape[1]
    bm = batch_mask.astype(jnp.int32)
    starts = jnp.searchsorted(bm, bm, side="left").astype(jnp.int32)
    ends = jnp.searchsorted(bm, bm, side="right").astype(jnp.int32)

    r0 = covalent_bonds[0].astype(jnp.int32)
    c0 = covalent_bonds[1].astype(jnp.int32)
    keys = r0 * N + c0
    korder = jnp.argsort(keys)
    sk = keys[korder]
    first_sorted = jnp.concatenate([jnp.ones((1,), bool), sk[1:] != sk[:-1]])
    is_first = jnp.zeros((nb,), bool).at[korder].set(first_sorted)
    minus_mask = is_first & (bm[r0] == bm[c0]) & (r0 != c0)
    midx = jnp.nonzero(minus_mask, size=MPAD, fill_value=0)[0]
    mvalid = jnp.arange(MPAD) < jnp.count_nonzero(minus_mask)

    npad = NE - nb - MPAD
    re = jnp.concatenate([r0, r0[midx] * mvalid, jnp.zeros((npad,), jnp.int32)])
    ce = jnp.concatenate([c0, c0[midx] * mvalid, jnp.zeros((npad,), jnp.int32)])
    sg = jnp.concatenate([jnp.ones((nb,), jnp.float32),
                          jnp.where(mvalid, -1.0, 0.0),
                          jnp.zeros((npad,), jnp.float32)])
    bte0 = jnp.concatenate(
        [bond_types.astype(jnp.float32)
         - jax.nn.one_hot(0, EF, dtype=jnp.float32)[None],
         jnp.zeros((NE - nb, EF), jnp.float32)])

    isd = (sg == 0.0).astype(jnp.int32)
    drank = jnp.cumsum(isd) - 1
    ndum = jnp.maximum(jnp.sum(isd), 1)
    refake = jnp.where(sg != 0.0, re, (drank * N) // ndum)
    dorder = jnp.argsort(refake)
    sdst = refake[dorder]
    bstart = jnp.searchsorted(
        sdst, jnp.arange(NWORK, dtype=jnp.int32) * ROWS).astype(jnp.int32)
    bend = jnp.concatenate([bstart[1:], jnp.full((1,), NE, jnp.int32)])
    jj = jnp.arange(NSLOT, dtype=jnp.int32)
    posr = bstart[:, None] + jj[None, :]
    valid = posr < bend[:, None]
    eidx = dorder[jnp.minimum(posr, NE - 1)]
    wbase = jnp.arange(NWORK, dtype=jnp.int32)[:, None] * ROWS
    rslot = jnp.where(valid, refake[eidx] - wbase, 0)
    gslot = jnp.where(valid, refake[eidx], 0)
    cslot = jnp.where(valid, ce[eidx], 0)
    sgslot = jnp.where(valid, sg[eidx], 0.0)
    rs3 = rslot.reshape(NWORK, NCHS, CH)
    gn3 = gslot.reshape(NWORK, NCHS, CH)
    cn3 = cslot.reshape(NWORK, NCHS, CH)
    sgb = jnp.broadcast_to(sgslot.reshape(NE2)[:, None], (NE2, 16))
    bte = bte0[eidx.reshape(NE2)]

    zrows = jnp.zeros((ROWS, NF), jnp.float32)
    cnt0 = ((ends - starts - 1).astype(jnp.float32)
            + jax.ops.segment_sum(sg, re, num_segments=N)).reshape(N, 1)
    starts2 = starts.reshape(N, 1)
    ends2 = ends.reshape(N, 1)

    i8 = jnp.arange(NBLK, dtype=jnp.int32) * BI
    jlo = (starts[i8] // 8) * 8
    nch = (ends[i8 + BI - 1] - jlo + BJ - 1) // BJ

    layers = params["layers"]
    wc3 = jnp.stack([lp["edge_mlp"]["l0"]["W"][:, 2 * NF:] for lp in layers])
    tq3 = _tq_call(bte, wc3)

    hcur = _emb_call(h.astype(jnp.float32),
                     params["emb_in"][0]["W"], params["emb_in"][0]["b"],
                     params["emb_in"][1]["W"], params["emb_in"][1]["b"])

    def stk(fn):
        return jnp.stack([fn(lp) for lp in layers])

    xs = dict(
        gn0=stk(lambda lp: lp["norm"]["g"]),
        bn0=stk(lambda lp: lp["norm"]["b"]),
        w0a=stk(lambda lp: lp["edge_mlp"]["l0"]["W"][:, :NF]),
        w0b=stk(lambda lp: lp["edge_mlp"]["l0"]["W"][:, NF:2 * NF]),
        cv=stk(lambda lp: lp["edge_mlp"]["l0"]["W"][:, 2 * NF]
               + lp["edge_mlp"]["l0"]["b"]),
        g1=stk(lambda lp: lp["edge_mlp"]["ln"]["g"]),
        b1=stk(lambda lp: lp["edge_mlp"]["ln"]["b"]),
        w1=stk(lambda lp: lp["edge_mlp"]["l1"]["W"]),
        b1e=stk(lambda lp: lp["edge_mlp"]["l1"]["b"]),
        wna=stk(lambda lp: lp["node_mlp"]["l0"]["W"][:, :NF]),
        wnb=stk(lambda lp: lp["node_mlp"]["l0"]["W"][:, NF:]),
        bn0e=stk(lambda lp: lp["node_mlp"]["l0"]["b"]),
        gnn=stk(lambda lp: lp["node_mlp"]["ln"]["g"]),
        bnn=stk(lambda lp: lp["node_mlp"]["ln"]["b"]),
        wn1=stk(lambda lp: lp["node_mlp"]["l1"]["W"]),
        bn1=stk(lambda lp: lp["node_mlp"]["l1"]["b"]),
        tql=tq3,
    )

    def layer_step(hc, p):
        hn, u, v, sact = _pre_call(
            hc, p["gn0"], p["bn0"], p["w0a"], p["w0b"], p["cv"],
            p["g1"], p["b1"])
        vpad = jnp.concatenate([v, jnp.zeros((BJ, NF), jnp.float32)])
        sden = _dense_call(jlo, nch, u, starts2, ends2, p["g1"], p["b1"],
                           vpad)
        lnp2 = jnp.stack([p["g1"], p["b1"]])
        acc = _sc_edges(u, v, p["tql"], rs3, gn3, cn3, sgb, lnp2, zrows)
        hout = _post_call(
            hn, sden, sact, acc[:N], cnt0, p["w1"], p["b1e"],
            p["wna"], p["wnb"], p["bn0e"], p["gnn"], p["bnn"],
            p["wn1"], p["bn1"])
        return hout, None

    hcur, _ = lax.scan(layer_step, hcur, xs)

    return _emb_call(hcur,
                     params["emb_out"][0]["W"], params["emb_out"][0]["b"],
                     params["emb_out"][1]["W"], params["emb_out"][1]["b"])


# speedup vs baseline: 1.0698x; 1.0698x over previous
import functools

import jax
import jax.numpy as jnp
from jax import lax
from jax.experimental import pallas as pl
from jax.experimental.pallas import tpu as pltpu
from jax.experimental.pallas import tpu_sc as plsc

N = 10000
NF = 128
EF = 5
MPAD = 4096
NE = 24576
NCORE = 2
NWORK = 16 * NCORE
ROWS = 320
NSLOT = 1024
CH = 128
NCHS = NSLOT // CH
NE2 = NWORK * NSLOT
BI = 8
BJ = 64
NBLK = N // BI
NROW = 1000
NOUT = NWORK * ROWS


def _silu(x):
    return x * (1.0 / (1.0 + jnp.exp(-x)))


def _lnorm(x, g, b):
    m = jnp.mean(x, axis=-1, keepdims=True)
    c = x - m
    v = jnp.mean(c * c, axis=-1, keepdims=True)
    return c * lax.rsqrt(v + 1e-5) * g + b


def _mmT(x, w):
    return lax.dot_general(x, w, (((x.ndim - 1,), (1,)), ((), ())),
                           preferred_element_type=jnp.float32)


def _emb_body(h_ref, w0_ref, b0_ref, w1_ref, b1_ref, o_ref):
    t = _silu(_mmT(h_ref[...], w0_ref[...]) + b0_ref[...])
    o_ref[...] = _mmT(t, w1_ref[...]) + b1_ref[...]


def _emb_call(h, w0, b0, w1, b1):
    wspec = pl.BlockSpec((NF, NF), lambda i: (0, 0))
    bspec = pl.BlockSpec((1, NF), lambda i: (0, 0))
    return pl.pallas_call(
        _emb_body,
        grid=(N // NROW,),
        in_specs=[pl.BlockSpec((NROW, NF), lambda i: (i, 0)),
                  wspec, bspec, wspec, bspec],
        out_specs=pl.BlockSpec((NROW, NF), lambda i: (i, 0)),
        out_shape=jax.ShapeDtypeStruct((N, NF), jnp.float32),
    )(h, w0, b0.reshape(1, NF), w1, b1.reshape(1, NF))


def _tq_body(bt_ref, wc_ref, o_ref):
    o_ref[...] = lax.dot_general(
        bt_ref[...], wc_ref[...][0], (((1,), (1,)), ((), ())),
        preferred_element_type=jnp.float32)[None]


def _tq_call(bte, wc3):
    nl = wc3.shape[0]
    blk = 2048
    return pl.pallas_call(
        _tq_body,
        grid=(nl, NE2 // blk),
        in_specs=[pl.BlockSpec((blk, EF), lambda l, j: (j, 0)),
                  pl.BlockSpec((1, NF, EF), lambda l, j: (l, 0, 0))],
        out_specs=pl.BlockSpec((1, blk, NF), lambda l, j: (l, j, 0)),
        out_shape=jax.ShapeDtypeStruct((nl, NE2, NF), jnp.float32),
    )(bte, wc3)


def _pre_body(h_ref, g_ref, b_ref, wa_ref, wb_ref, cv_ref, g1_ref, b1_ref,
              hn_ref, u_ref, v_ref, sact_ref):
    hn = _lnorm(h_ref[...], g_ref[...], b_ref[...])
    u = _mmT(hn, wa_ref[...]) + cv_ref[...]
    v = _mmT(hn, wb_ref[...])
    hn_ref[...] = hn
    u_ref[...] = u
    v_ref[...] = v
    sact_ref[...] = _silu(_lnorm(u + v, g1_ref[...], b1_ref[...]))


def _pre_call(h, g, b, wa, wb, cv, g1, b1):
    wspec = pl.BlockSpec((NF, NF), lambda i: (0, 0))
    bspec = pl.BlockSpec((1, NF), lambda i: (0, 0))
    rspec = pl.BlockSpec((NROW, NF), lambda i: (i, 0))
    sd = jax.ShapeDtypeStruct((N, NF), jnp.float32)
    return pl.pallas_call(
        _pre_body,
        grid=(N // NROW,),
        in_specs=[rspec, bspec, bspec, wspec, wspec, bspec, bspec, bspec],
        out_specs=[rspec, rspec, rspec, rspec],
        out_shape=[sd, sd, sd, sd],
    )(h, g.reshape(1, NF), b.reshape(1, NF), wa, wb,
      cv.reshape(1, NF), g1.reshape(1, NF), b1.reshape(1, NF))


def _dense_body(jlo_ref, nch_ref, u_ref, s_ref, e_ref, g1_ref, b1_ref,
                vfull_ref, out_ref):
    i = pl.program_id(0)
    j0 = jlo_ref[i]
    nch = nch_ref[i]
    ub = u_ref[...][:, None, :]
    srow = s_ref[...]
    erow = e_ref[...]
    g1 = g1_ref[...][None]
    b1 = b1_ref[...][None]

    def chunk(k, acc):
        jb = j0 + k * BJ
        vblk = vfull_ref[pl.ds(jb, BJ), :]
        z = ub + vblk[None, :, :]
        m = jnp.mean(z, axis=-1, keepdims=True)
        c = z - m
        var = jnp.mean(c * c, axis=-1, keepdims=True)
        a = _silu(c * lax.rsqrt(var + 1e-5) * g1 + b1)
        jid3 = jb + lax.broadcasted_iota(jnp.int32, (BI, BJ, 1), 1)
        msk3 = (jid3 >= srow[:, None, :]) & (jid3 < erow[:, None, :])
        a = jnp.where(msk3, a, 0.0)
        return acc + jnp.sum(a, axis=1)

    out_ref[...] = lax.fori_loop(0, nch, chunk,
                                 jnp.zeros((BI, NF), jnp.float32))


def _dense_call(jlo, nch, u, starts2, ends2, g1, b1, vpad):
    grid_spec = pltpu.PrefetchScalarGridSpec(
        num_scalar_prefetch=2,
        grid=(NBLK,),
        in_specs=[pl.BlockSpec((BI, NF), lambda i, *_: (i, 0)),
                  pl.BlockSpec((BI, 1), lambda i, *_: (i, 0)),
                  pl.BlockSpec((BI, 1), lambda i, *_: (i, 0)),
                  pl.BlockSpec((1, NF), lambda i, *_: (0, 0)),
                  pl.BlockSpec((1, NF), lambda i, *_: (0, 0)),
                  pl.BlockSpec(vpad.shape, lambda i, *_: (0, 0))],
        out_specs=pl.BlockSpec((BI, NF), lambda i, *_: (i, 0)),
    )
    return pl.pallas_call(
        _dense_body,
        grid_spec=grid_spec,
        out_shape=jax.ShapeDtypeStruct((N, NF), jnp.float32),
    )(jlo, nch, u, starts2, ends2, g1.reshape(1, NF), b1.reshape(1, NF), vpad)


def _post_body(hn_ref, sden_ref, sact_ref, acc_ref, cnt_ref, w1_ref, b1_ref,
               wna_ref, wnb_ref, bn0_ref, gn_ref, bn_ref, wn1_ref, bn1_ref,
               out_ref):
    atot = sden_ref[...] - sact_ref[...] + acc_ref[...]
    agg = _mmT(atot, w1_ref[...]) + cnt_ref[...] * b1_ref[...]
    hn = hn_ref[...]
    a = _mmT(hn, wna_ref[...]) + _mmT(agg, wnb_ref[...]) + bn0_ref[...]
    a = _silu(_lnorm(a, gn_ref[...], bn_ref[...]))
    out_ref[...] = hn + _mmT(a, wn1_ref[...]) + bn1_ref[...]


def _post_call(hn, sden, sact, acc, cnt, w1, b1, wna, wnb, bn0, gn, bn,
               wn1, bn1):
    wspec = pl.BlockSpec((NF, NF), lambda i: (0, 0))
    bspec = pl.BlockSpec((1, NF), lambda i: (0, 0))
    rspec = pl.BlockSpec((NROW, NF), lambda i: (i, 0))
    return pl.pallas_call(
        _post_body,
        grid=(N // NROW,),
        in_specs=[rspec, rspec, rspec, rspec,
                  pl.BlockSpec((NROW, 1), lambda i: (i, 0)),
                  wspec, bspec, wspec, wspec, bspec, bspec, bspec,
                  wspec, bspec],
        out_specs=rspec,
        out_shape=jax.ShapeDtypeStruct((N, NF), jnp.float32),
    )(hn, sden, sact, acc, cnt, w1, b1.reshape(1, NF), wna, wnb,
      bn0.reshape(1, NF), gn.reshape(1, NF), bn.reshape(1, NF),
      wn1, bn1.reshape(1, NF))


def _allsum16(x):
    for sh in (8, 4, 2, 1):
        idx = (lax.iota(jnp.int32, 16) ^ sh)[:, None]
        g = lax.gather(
            x, idx,
            lax.GatherDimensionNumbers(offset_dims=(),
                                       collapsed_slice_dims=(0,),
                                       start_index_map=(0,)),
            (1,), mode=lax.GatherScatterMode.PROMISE_IN_BOUNDS)
        x = x + g
    return x


def _sc_edges_body(u_hbm, v_hbm, tq_hbm, rs_hbm, gn_hbm, cn_hbm, sgb_hbm,
                   lnp_hbm, zer_hbm, out_hbm,
                   rsv, gnv, cnv, urows, vrows, tqv, sgv, lnv, obuf, table,
                   sem1, sem2):
    cid = lax.axis_index("c")
    sid = lax.axis_index("s")
    wid = sid * NCORE + cid
    base = wid * NSLOT

    pltpu.sync_copy(rs_hbm.at[wid], rsv)
    pltpu.sync_copy(gn_hbm.at[wid], gnv)
    pltpu.sync_copy(cn_hbm.at[wid], cnv)
    pltpu.sync_copy(lnp_hbm, lnv)
    pltpu.sync_copy(zer_hbm, table)

    def chunk(k, c0):
        cp1 = pltpu.async_copy(u_hbm.at[gnv.at[k]], urows, sem1)
        cp2 = pltpu.async_copy(v_hbm.at[cnv.at[k]], vrows, sem2)
        pltpu.sync_copy(tq_hbm.at[pl.ds(base + k * CH, CH)], tqv)
        pltpu.sync_copy(sgb_hbm.at[pl.ds(base + k * CH, CH)], sgv)
        cp1.wait()
        cp2.wait()

        def comp(g):
            for e16 in range(16):
                e = g * 16 + e16
                zq = [urows[e, pl.ds(16 * q, 16)]
                      + vrows[e, pl.ds(16 * q, 16)]
                      + tqv[e, pl.ds(16 * q, 16)] for q in range(8)]
                s = zq[0]
                for q in range(1, 8):
                    s = s + zq[q]
                m = _allsum16(s) * (1.0 / 128.0)
                cq = [zq[q] - m for q in range(8)]
                ss = cq[0] * cq[0]
                for q in range(1, 8):
                    ss = ss + cq[q] * cq[q]
                xv = _allsum16(ss) * (1.0 / 128.0) + 1e-5
                ib = lax.bitcast_convert_type(xv, jnp.int32)
                y = lax.bitcast_convert_type(
                    jnp.int32(0x5F3759DF) - (ib >> 1), jnp.float32)
                xh = xv * 0.5
                for _ in range(3):
                    y = y * (1.5 - xh * y * y)
                sg = sgv[e, :]
                for q in range(8):
                    sl = pl.ds(16 * q, 16)
                    yn = cq[q] * y * lnv[0, sl] + lnv[1, sl]
                    sil = yn * (1.0 / (1.0 + jnp.exp(-yn)))
                    obuf[e, sl] = sil * sg

        plsc.parallel_loop(0, CH // 16)(comp)

        def accum(g, c1):
            r16 = rsv[k, pl.ds(16 * g, 16)]
            for e16 in range(16):
                e = g * 16 + e16
                row = r16[e16]
                for q in range(8):
                    sl = pl.ds(16 * q, 16)
                    table[row, sl] = table[row, sl] + obuf[e, sl]
            return c1

        lax.fori_loop(0, CH // 16, accum, 0)
        return c0

    lax.fori_loop(0, NCHS, chunk, 0)
    pltpu.sync_copy(table, out_hbm.at[pl.ds(wid * ROWS, ROWS)])


@functools.cache
def _make_sc_edges():
    return functools.partial(
        pl.kernel,
        mesh=plsc.VectorSubcoreMesh(core_axis_name="c", subcore_axis_name="s",
                                    num_cores=NCORE),
        compiler_params=pltpu.CompilerParams(needs_layout_passes=False),
        out_type=jax.ShapeDtypeStruct((NOUT, NF), jnp.float32),
        scratch_types=[
            pltpu.VMEM((NCHS, CH), jnp.int32),
            pltpu.VMEM((NCHS, CH), jnp.int32),
            pltpu.VMEM((NCHS, CH), jnp.int32),
            pltpu.VMEM((CH, NF), jnp.float32),
            pltpu.VMEM((CH, NF), jnp.float32),
            pltpu.VMEM((CH, NF), jnp.float32),
            pltpu.VMEM((CH, 16), jnp.float32),
            pltpu.VMEM((2, NF), jnp.float32),
            pltpu.VMEM((CH, NF), jnp.float32),
            pltpu.VMEM((ROWS, NF), jnp.float32),
            pltpu.SemaphoreType.DMA,
            pltpu.SemaphoreType.DMA,
        ],
    )(_sc_edges_body)


def _sc_edges(*args):
    return _make_sc_edges()(*args)


def kernel(x, h, batch_mask, covalent_bonds, bond_types, mol_feats, params):
    nb = covalent_bonds.shape[1]
    bm = batch_mask.astype(jnp.int32)
    starts = jnp.searchsorted(bm, bm, side="left").astype(jnp.int32)
    ends = jnp.searchsorted(bm, bm, side="right").astype(jnp.int32)

    r0 = covalent_bonds[0].astype(jnp.int32)
    c0 = covalent_bonds[1].astype(jnp.int32)
    keys = r0 * N + c0
    korder = jnp.argsort(keys)
    sk = keys[korder]
    first_sorted = jnp.concatenate([jnp.ones((1,), bool), sk[1:] != sk[:-1]])
    is_first = jnp.zeros((nb,), bool).at[korder].set(first_sorted)
    minus_mask = is_first & (bm[r0] == bm[c0]) & (r0 != c0)
    midx = jnp.nonzero(minus_mask, size=MPAD, fill_value=0)[0]
    mvalid = jnp.arange(MPAD) < jnp.count_nonzero(minus_mask)

    npad = NE - nb - MPAD
    re = jnp.concatenate([r0, r0[midx] * mvalid, jnp.zeros((npad,), jnp.int32)])
    ce = jnp.concatenate([c0, c0[midx] * mvalid, jnp.zeros((npad,), jnp.int32)])
    sg = jnp.concatenate([jnp.ones((nb,), jnp.float32),
                          jnp.where(mvalid, -1.0, 0.0),
                          jnp.zeros((npad,), jnp.float32)])
    bte0 = jnp.concatenate(
        [bond_types.astype(jnp.float32)
         - jax.nn.one_hot(0, EF, dtype=jnp.float32)[None],
         jnp.zeros((NE - nb, EF), jnp.float32)])

    isd = (sg == 0.0).astype(jnp.int32)
    drank = jnp.cumsum(isd) - 1
    ndum = jnp.maximum(jnp.sum(isd), 1)
    refake = jnp.where(sg != 0.0, re, (drank * N) // ndum)
    dorder = jnp.argsort(refake)
    sdst = refake[dorder]
    bstart = jnp.searchsorted(
        sdst, jnp.arange(NWORK, dtype=jnp.int32) * ROWS).astype(jnp.int32)
    bend = jnp.concatenate([bstart[1:], jnp.full((1,), NE, jnp.int32)])
    jj = jnp.arange(NSLOT, dtype=jnp.int32)
    posr = bstart[:, None] + jj[None, :]
    valid = posr < bend[:, None]
    eidx = dorder[jnp.minimum(posr, NE - 1)]
    wbase = jnp.arange(NWORK, dtype=jnp.int32)[:, None] * ROWS
    rslot = jnp.where(valid, refake[eidx] - wbase, 0)
    gslot = jnp.where(valid, refake[eidx], 0)
    cslot = jnp.where(valid, ce[eidx], 0)
    sgslot = jnp.where(valid, sg[eidx], 0.0)
    rs3 = rslot.reshape(NWORK, NCHS, CH)
    gn3 = gslot.reshape(NWORK, NCHS, CH)
    cn3 = cslot.reshape(NWORK, NCHS, CH)
    sgb = jnp.broadcast_to(sgslot.reshape(NE2)[:, None], (NE2, 16))
    bte = bte0[eidx.reshape(NE2)]

    zrows = jnp.zeros((ROWS, NF), jnp.float32)
    cnt0 = ((ends - starts - 1).astype(jnp.float32)
            + jax.ops.segment_sum(sg, re, num_segments=N)).reshape(N, 1)
    starts2 = starts.reshape(N, 1)
    ends2 = ends.reshape(N, 1)

    i8 = jnp.arange(NBLK, dtype=jnp.int32) * BI
    jlo = (starts[i8] // 8) * 8
    nch = (ends[i8 + BI - 1] - jlo + BJ - 1) // BJ

    layers = params["layers"]
    wc3 = jnp.stack([lp["edge_mlp"]["l0"]["W"][:, 2 * NF:] for lp in layers])
    tq3 = _tq_call(bte, wc3)

    hcur = _emb_call(h.astype(jnp.float32),
                     params["emb_in"][0]["W"], params["emb_in"][0]["b"],
                     params["emb_in"][1]["W"], params["emb_in"][1]["b"])

    def stk(fn):
        return jnp.stack([fn(lp) for lp in layers])

    xs = dict(
        gn0=stk(lambda lp: lp["norm"]["g"]),
        bn0=stk(lambda lp: lp["norm"]["b"]),
        w0a=stk(lambda lp: lp["edge_mlp"]["l0"]["W"][:, :NF]),
        w0b=stk(lambda lp: lp["edge_mlp"]["l0"]["W"][:, NF:2 * NF]),
        cv=stk(lambda lp: lp["edge_mlp"]["l0"]["W"][:, 2 * NF]
               + lp["edge_mlp"]["l0"]["b"]),
        g1=stk(lambda lp: lp["edge_mlp"]["ln"]["g"]),
        b1=stk(lambda lp: lp["edge_mlp"]["ln"]["b"]),
        w1=stk(lambda lp: lp["edge_mlp"]["l1"]["W"]),
        b1e=stk(lambda lp: lp["edge_mlp"]["l1"]["b"]),
        wna=stk(lambda lp: lp["node_mlp"]["l0"]["W"][:, :NF]),
        wnb=stk(lambda lp: lp["node_mlp"]["l0"]["W"][:, NF:]),
        bn0e=stk(lambda lp: lp["node_mlp"]["l0"]["b"]),
        gnn=stk(lambda lp: lp["node_mlp"]["ln"]["g"]),
        bnn=stk(lambda lp: lp["node_mlp"]["ln"]["b"]),
        wn1=stk(lambda lp: lp["node_mlp"]["l1"]["W"]),
        bn1=stk(lambda lp: lp["node_mlp"]["l1"]["b"]),
        tql=tq3,
    )

    def layer_step(hc, p):
        hn, u, v, sact = _pre_call(
            hc, p["gn0"], p["bn0"], p["w0a"], p["w0b"], p["cv"],
            p["g1"], p["b1"])
        vpad = jnp.concatenate([v, jnp.zeros((BJ, NF), jnp.float32)])
        sden = _dense_call(jlo, nch, u, starts2, ends2, p["g1"], p["b1"],
                           vpad)
        lnp2 = jnp.stack([p["g1"], p["b1"]])
        acc = _sc_edges(u, v, p["tql"], rs3, gn3, cn3, sgb, lnp2, zrows)
        hout = _post_call(
            hn, sden, sact, acc[:N], cnt0, p["w1"], p["b1e"],
            p["wna"], p["wnb"], p["bn0e"], p["gnn"], p["bnn"],
            p["wn1"], p["bn1"])
        return hout, None

    hcur, _ = lax.scan(layer_step, hcur, xs)

    return _emb_call(hcur,
                     params["emb_out"][0]["W"], params["emb_out"][0]["b"],
                     params["emb_out"][1]["W"], params["emb_out"][1]["b"])
